# baseline (speedup 1.0000x reference)
import functools

import jax
import jax.numpy as jnp
from jax import lax
from jax.experimental import pallas as pl
from jax.experimental.pallas import tpu as pltpu
from jax.experimental.pallas import tpu_sc as plsc

N_EMBEDDINGS = 8192
EMBEDDING_DIM = 256
N_ROWS = 16384


_R = 256
_RB = N_ROWS // _R

_SPANS = ((0, 2736), (2736, 5472), (5472, N_EMBEDDINGS))
_INT_MAX = jnp.iinfo(jnp.int32).max


def _argmin_body(x_ref, c_ref, sq1_ref, sq2_ref, idx_ref, enc_ref):
    x = x_ref[...]
    sq1 = sq1_ref[0]
    acc_v = None
    acc_i = None
    for a, b in _SPANS:
        cs = c_ref[a:b, :]
        m = lax.dot_general(cs, x, (((1,), (1,)), ((), ())),
                            preferred_element_type=jnp.float32)
        sq2s = sq2_ref[a:b, :]
        d = (sq1 - 2.0 * m) + sq2s
        tmin = jnp.min(d, axis=0, keepdims=True)
        ids = lax.broadcasted_iota(jnp.int32, d.shape, 0) + a
        targ = jnp.min(jnp.where(d == tmin, ids, _INT_MAX),
                       axis=0, keepdims=True)
        if acc_v is None:
            acc_v, acc_i = tmin, targ
        else:
            upd = tmin < acc_v
            acc_v = jnp.where(upd, tmin, acc_v)
            acc_i = jnp.where(upd, targ, acc_i)
        acc_v = acc_v.astype(jnp.bfloat16).astype(jnp.float32)
    idx_ref[...] = acc_i[None]
    idx_t = acc_i.reshape(_R, 1)
    ids2 = lax.broadcasted_iota(jnp.int32, (_R, N_EMBEDDINGS), 1)
    enc_ref[...] = (ids2 == idx_t).astype(jnp.float32)


_argmin_call = pl.pallas_call(
    _argmin_body,
    grid=(_RB,),
    in_specs=[
        pl.BlockSpec((_R, EMBEDDING_DIM), lambda r: (r, 0)),
        pl.BlockSpec((N_EMBEDDINGS, EMBEDDING_DIM), lambda r: (0, 0)),
        pl.BlockSpec((1, 1, _R), lambda r: (r, 0, 0)),
        pl.BlockSpec((N_EMBEDDINGS, 1), lambda r: (0, 0)),
    ],
    out_specs=[
        pl.BlockSpec((1, 1, _R), lambda r: (r, 0, 0)),
        pl.BlockSpec((_R, N_EMBEDDINGS), lambda r: (r, 0)),
    ],
    out_shape=[
        jax.ShapeDtypeStruct((_RB, 1, _R), jnp.int32),
        jax.ShapeDtypeStruct((N_ROWS, N_EMBEDDINGS), jnp.float32),
    ],
    compiler_params=pltpu.CompilerParams(
        dimension_semantics=("arbitrary",)),
)


_R2 = 512
_C2 = 2048
_RB2 = N_ROWS // _R2
_CB2 = N_EMBEDDINGS // _C2


def _onehot_body(idx_ref, out_ref):
    cb = pl.program_id(1)
    ids = lax.broadcasted_iota(jnp.int32, out_ref.shape, 1) + cb * _C2
    out_ref[...] = (ids == idx_ref[...]).astype(jnp.float32)


_onehot_call = pl.pallas_call(
    _onehot_body,
    grid=(_RB2, _CB2),
    in_specs=[pl.BlockSpec((_R2, 1), lambda r, c: (r, 0))],
    out_specs=pl.BlockSpec((_R2, _C2), lambda r, c: (r, c)),
    out_shape=jax.ShapeDtypeStruct((N_ROWS, N_EMBEDDINGS), jnp.float32),
    compiler_params=pltpu.CompilerParams(
        dimension_semantics=("parallel", "parallel")),
)


_NC = 2
_NS = 16
_NW = _NC * _NS
_B_PER_W = N_ROWS // _NW
_CHUNK = 128


@functools.lru_cache(maxsize=1)
def _make_gather():
    mesh = plsc.VectorSubcoreMesh(core_axis_name="c", subcore_axis_name="s")

    @functools.partial(
        pl.kernel,
        mesh=mesh,
        out_type=jax.ShapeDtypeStruct((N_ROWS, EMBEDDING_DIM), jnp.float32),
        scratch_types=[
            pltpu.VMEM((_CHUNK,), jnp.int32),
            pltpu.VMEM((_CHUNK, EMBEDDING_DIM), jnp.float32),
            pltpu.SemaphoreType.DMA,
        ],
    )
    def _gather(table_hbm, idx_hbm, out_hbm, idx_v, rows_v, sem):
        wid = lax.axis_index("s") * _NC + lax.axis_index("c")
        base = wid * _B_PER_W
        for chunk in range(_B_PER_W // _CHUNK):
            off = base + chunk * _CHUNK
            pltpu.sync_copy(idx_hbm.at[pl.ds(off, _CHUNK)], idx_v)
            pltpu.async_copy(table_hbm.at[idx_v], rows_v, sem).wait()
            pltpu.sync_copy(rows_v, out_hbm.at[pl.ds(off, _CHUNK)])

    return _gather


def kernel(inputs, codebook):
    shape = inputs.shape
    flat = inputs.reshape(-1, EMBEDDING_DIM)
    sq1 = jnp.sum(flat ** 2, axis=1)
    sq2 = jnp.sum(codebook ** 2, axis=1)
    idx3, enc = _argmin_call(flat, codebook,
                             sq1.reshape(_RB, 1, _R),
                             sq2.reshape(N_EMBEDDINGS, 1))
    q = _make_gather()(codebook, idx3.reshape(-1))
    return (q.reshape(shape), enc)


# speedup vs baseline: 1.0688x; 1.0688x over previous
import functools

import jax
import jax.numpy as jnp
from jax import lax
from jax.experimental import pallas as pl
from jax.experimental.pallas import tpu as pltpu
from jax.experimental.pallas import tpu_sc as plsc

N_EMBEDDINGS = 8192
EMBEDDING_DIM = 256
N_ROWS = 16384


_R = 512
_RB = N_ROWS // _R

_SPANS = ((0, 2736), (2736, 5472), (5472, N_EMBEDDINGS))
_INT_MAX = jnp.iinfo(jnp.int32).max


def _argmin_body(x_ref, c_ref, sq1_ref, sq2_ref, idx_ref, enc_ref):
    x = x_ref[...]
    sq1 = sq1_ref[0]
    acc_v = None
    acc_i = None
    for a, b in _SPANS:
        cs = c_ref[a:b, :]
        m = lax.dot_general(cs, x, (((1,), (1,)), ((), ())),
                            preferred_element_type=jnp.float32)
        sq2s = sq2_ref[a:b, :]
        d = (sq1 - 2.0 * m) + sq2s
        tmin = jnp.min(d, axis=0, keepdims=True)
        ids = lax.broadcasted_iota(jnp.int32, d.shape, 0) + a
        targ = jnp.min(jnp.where(d == tmin, ids, _INT_MAX),
                       axis=0, keepdims=True)
        if acc_v is None:
            acc_v, acc_i = tmin, targ
        else:
            upd = tmin < acc_v
            acc_v = jnp.where(upd, tmin, acc_v)
            acc_i = jnp.where(upd, targ, acc_i)
        acc_v = acc_v.astype(jnp.bfloat16).astype(jnp.float32)
    idx_ref[...] = acc_i[None]
    idx_t = acc_i.reshape(_R, 1)
    ids2 = lax.broadcasted_iota(jnp.int32, (_R, N_EMBEDDINGS), 1)
    enc_ref[...] = (ids2 == idx_t).astype(jnp.float32)


_argmin_call = pl.pallas_call(
    _argmin_body,
    grid=(_RB,),
    in_specs=[
        pl.BlockSpec((_R, EMBEDDING_DIM), lambda r: (r, 0)),
        pl.BlockSpec((N_EMBEDDINGS, EMBEDDING_DIM), lambda r: (0, 0)),
        pl.BlockSpec((1, 1, _R), lambda r: (r, 0, 0)),
        pl.BlockSpec((N_EMBEDDINGS, 1), lambda r: (0, 0)),
    ],
    out_specs=[
        pl.BlockSpec((1, 1, _R), lambda r: (r, 0, 0)),
        pl.BlockSpec((_R, N_EMBEDDINGS), lambda r: (r, 0)),
    ],
    out_shape=[
        jax.ShapeDtypeStruct((_RB, 1, _R), jnp.int32),
        jax.ShapeDtypeStruct((N_ROWS, N_EMBEDDINGS), jnp.float32),
    ],
    compiler_params=pltpu.CompilerParams(
        dimension_semantics=("arbitrary",)),
)


_NC = 2
_NS = 16
_NW = _NC * _NS
_B_PER_W = N_ROWS // _NW
_CHUNK = 128


@functools.lru_cache(maxsize=1)
def _make_gather():
    mesh = plsc.VectorSubcoreMesh(core_axis_name="c", subcore_axis_name="s")

    @functools.partial(
        pl.kernel,
        mesh=mesh,
        out_type=jax.ShapeDtypeStruct((N_ROWS, EMBEDDING_DIM), jnp.float32),
        scratch_types=[
            pltpu.VMEM((_CHUNK,), jnp.int32),
            pltpu.VMEM((_CHUNK, EMBEDDING_DIM), jnp.float32),
            pltpu.SemaphoreType.DMA,
        ],
    )
    def _gather(table_hbm, idx_hbm, out_hbm, idx_v, rows_v, sem):
        wid = lax.axis_index("s") * _NC + lax.axis_index("c")
        base = wid * _B_PER_W
        for chunk in range(_B_PER_W // _CHUNK):
            off = base + chunk * _CHUNK
            pltpu.sync_copy(idx_hbm.at[pl.ds(off, _CHUNK)], idx_v)
            pltpu.async_copy(table_hbm.at[idx_v], rows_v, sem).wait()
            pltpu.sync_copy(rows_v, out_hbm.at[pl.ds(off, _CHUNK)])

    return _gather


def kernel(inputs, codebook):
    shape = inputs.shape
    flat = inputs.reshape(-1, EMBEDDING_DIM)
    sq1 = jnp.sum(flat ** 2, axis=1)
    sq2 = jnp.sum(codebook ** 2, axis=1)
    idx3, enc = _argmin_call(flat, codebook,
                             sq1.reshape(_RB, 1, _R),
                             sq2.reshape(N_EMBEDDINGS, 1))
    q = _make_gather()(codebook, idx3.reshape(-1))
    return (q.reshape(shape), enc)
